# baseline (speedup 1.0000x reference)
import functools

import jax
import jax.numpy as jnp
from jax import lax
from jax.experimental import pallas as pl
from jax.experimental.pallas import tpu as pltpu
from jax.experimental.pallas import tpu_sc as plsc

_N, _D, _K = 4096, 512, 512
_R, _C = 32, 128


def _fused_body(e_ref, w_ref, mrow_ref, b_ref, idx_ref, s_ref, mo_ref,
                s_scr):
    i = pl.program_id(0)
    s_scr[pl.ds(i * _BLK, _BLK), :] = jnp.dot(
        e_ref[...], w_ref[...], preferred_element_type=jnp.float32)

    @pl.when(i == _N // _BLK - 1)
    def _select_phase():
        _select_tail(s_scr, mrow_ref, b_ref, idx_ref, s_ref, mo_ref)


def _select_tail(s_scr, mrow_ref, b_ref, idx_ref, s_ref, mo_ref):
    b = b_ref[0, 0]
    s = s_scr[...]
    srow0 = s.reshape(1, _N)
    xs_blk = s.reshape(_R, _C).T
    m_blk = mrow_ref[...].reshape(_R, _C).T

    xs = (xs_blk + b) + jnp.log(m_blk)

    ui = lax.bitcast_convert_type(xs, jnp.uint32)
    key = ui ^ jnp.where(ui >= jnp.uint32(0x80000000),
                         jnp.uint32(0xFFFFFFFF), jnp.uint32(0x80000000))

    def bs_body(_, carry):
        t, bit = carry
        cand = t | bit
        cnt = jnp.sum((key >= cand).astype(jnp.int32))
        return jnp.where(cnt >= _K, cand, t), bit >> jnp.uint32(1)

    thr, _ = lax.fori_loop(0, 32, bs_body,
                           (jnp.uint32(0), jnp.uint32(0x80000000)))

    gt = key > thr
    eq = key == thr
    m_ties = (_K - jnp.sum(gt.astype(jnp.int32))).astype(jnp.float32)

    ci0 = lax.broadcasted_iota(jnp.int32, (_C, _C), 0)
    ci1 = lax.broadcasted_iota(jnp.int32, (_C, _C), 1)
    tri_incl = (ci0 >= ci1).astype(jnp.float32)
    ones_c = jnp.ones((_C, _C), jnp.float32)
    ri0 = lax.broadcasted_iota(jnp.int32, (_R, _R), 0)
    ri1 = lax.broadcasted_iota(jnp.int32, (_R, _R), 1)
    su = (ri0 < ri1).astype(jnp.float32)

    def icumsum(x):
        tot = jnp.dot(ones_c, x, preferred_element_type=jnp.float32)
        return (jnp.dot(tri_incl, x, preferred_element_type=jnp.float32)
                + jnp.dot(tot, su, preferred_element_type=jnp.float32))

    eqrank = icumsum(eq.astype(jnp.float32))
    keep = gt | (eq & (eqrank <= m_ties))
    kf = keep.astype(jnp.float32)
    pose = icumsum(kf) - kf

    prow = lax.broadcasted_iota(jnp.int32, (_C, _K), 1).astype(jnp.float32)
    pose2 = pose + (1.0 - kf) * jnp.float32(_N)
    oh_blocks = []
    for r in range(_R):
        oh_blocks.append(
            (pose2[:, r:r + 1] == prow).astype(jnp.bfloat16))
    oh_full = jnp.concatenate(oh_blocks, axis=0)

    iv = lax.broadcasted_iota(jnp.int32, (1, _N), 1)
    civ_hi = (iv >> 8).astype(jnp.float32)
    civ_lo = (iv & 255).astype(jnp.float32)
    srow = (srow0 + b) + jnp.log(mrow_ref[...])
    srow = jnp.maximum(srow, jnp.float32(-3.0e38))

    def split3(x):
        x1 = x.astype(jnp.bfloat16).astype(jnp.float32)
        r1 = x - x1
        x2 = r1.astype(jnp.bfloat16).astype(jnp.float32)
        return x1, x2, r1 - x2

    s1, s2, s3 = split3(srow)
    m1, m2, m3 = split3(mrow_ref[...])
    vals = jnp.concatenate([civ_hi, civ_lo, s1, s2, s3, m1, m2, m3],
                           axis=0).astype(jnp.bfloat16)
    out8 = jnp.dot(vals, oh_full, preferred_element_type=jnp.float32)
    idx_ref[...] = (out8[0:1] * 256.0 + out8[1:2]).astype(jnp.int32)
    s_ref[...] = (out8[2:3] + out8[3:4]) + out8[4:5]
    mo_ref[...] = (out8[5:6] + out8[6:7]) + out8[7:8]


_BLK = 2048

_fused = pl.pallas_call(
    _fused_body,
    grid=(_N // _BLK,),
    in_specs=[pl.BlockSpec((_BLK, _D), lambda i: (i, 0)),
              pl.BlockSpec((_D, 1), lambda i: (0, 0)),
              pl.BlockSpec((1, _N), lambda i: (0, 0)),
              pl.BlockSpec((1, 1), lambda i: (0, 0))],
    out_specs=[pl.BlockSpec((1, _K), lambda i: (0, 0)),
               pl.BlockSpec((1, _K), lambda i: (0, 0)),
               pl.BlockSpec((1, _K), lambda i: (0, 0))],
    out_shape=[jax.ShapeDtypeStruct((1, _K), jnp.int32),
               jax.ShapeDtypeStruct((1, _K), jnp.float32),
               jax.ShapeDtypeStruct((1, _K), jnp.float32)],
    scratch_shapes=[pltpu.VMEM((_N, 1), jnp.float32)],
)


def _gather_body(e_hbm, idx_hbm, emb_out, idxv, rows, sem):
    wid = lax.axis_index("s") * 2 + lax.axis_index("c")
    base = wid * 16
    pltpu.sync_copy(idx_hbm.at[pl.ds(base, 16)], idxv)
    pltpu.async_copy(e_hbm.at[idxv], rows, sem).wait()
    pltpu.sync_copy(rows, emb_out.at[pl.ds(base, 16)])


@functools.cache
def _make_gather():
    return pl.kernel(
        _gather_body,
        mesh=plsc.VectorSubcoreMesh(core_axis_name="c", subcore_axis_name="s"),
        out_type=jax.ShapeDtypeStruct((_K, _D), jnp.float32),
        scratch_types=[pltpu.VMEM((16,), jnp.int32),
                       pltpu.VMEM((16, _D), jnp.float32),
                       pltpu.SemaphoreType.DMA],
    )


def kernel(span_embeddings, spans, span_mask, num_spans_to_keep, W, b):
    del spans, num_spans_to_keep
    e2 = span_embeddings.reshape(_N, _D)
    mrow = span_mask.reshape(1, _N)
    idx2, tops, topm = _fused(e2, W, mrow, b.reshape(1, 1))
    emb = _make_gather()(e2, idx2.reshape(_K))
    return (emb[None], topm, idx2, tops[0][None, :, None])


# speedup vs baseline: 1.0023x; 1.0023x over previous
import functools

import jax
import jax.numpy as jnp
from jax import lax
from jax.experimental import pallas as pl
from jax.experimental.pallas import tpu as pltpu
from jax.experimental.pallas import tpu_sc as plsc

_N, _D, _K = 4096, 512, 512
_R, _C = 32, 128


def _fused_body(e_ref, w_ref, mrow_ref, b_ref, idx_ref, s_ref, mo_ref,
                s_scr):
    i = pl.program_id(0)
    s_scr[pl.ds(i * _BLK, _BLK), :] = jnp.dot(
        e_ref[...], w_ref[...], preferred_element_type=jnp.float32)

    @pl.when(i == _N // _BLK - 1)
    def _select_phase():
        _select_tail(s_scr, mrow_ref, b_ref, idx_ref, s_ref, mo_ref)


def _select_tail(s_scr, mrow_ref, b_ref, idx_ref, s_ref, mo_ref):
    b = b_ref[0, 0]
    s = s_scr[...]
    srow0 = s.reshape(1, _N)
    xs_blk = s.reshape(_R, _C).T
    m_blk = mrow_ref[...].reshape(_R, _C).T

    xs = (xs_blk + b) + jnp.log(m_blk)

    ui = lax.bitcast_convert_type(xs, jnp.uint32)
    key = ui ^ jnp.where(ui >= jnp.uint32(0x80000000),
                         jnp.uint32(0xFFFFFFFF), jnp.uint32(0x80000000))

    def bs_body(_, carry):
        t, bit = carry
        cand = t | bit
        cnt = jnp.sum((key >= cand).astype(jnp.int32))
        return jnp.where(cnt >= _K, cand, t), bit >> jnp.uint32(1)

    thr, _ = lax.fori_loop(0, 32, bs_body,
                           (jnp.uint32(0), jnp.uint32(0x80000000)))

    gt = key > thr
    eq = key == thr
    m_ties = (_K - jnp.sum(gt.astype(jnp.int32))).astype(jnp.float32)

    ci0 = lax.broadcasted_iota(jnp.int32, (_C, _C), 0)
    ci1 = lax.broadcasted_iota(jnp.int32, (_C, _C), 1)
    tri_incl = (ci0 >= ci1).astype(jnp.float32)
    ones_c = jnp.ones((_C, _C), jnp.float32)
    ri0 = lax.broadcasted_iota(jnp.int32, (_R, _R), 0)
    ri1 = lax.broadcasted_iota(jnp.int32, (_R, _R), 1)
    su = (ri0 < ri1).astype(jnp.float32)

    def icumsum(x):
        tot = jnp.dot(ones_c, x, preferred_element_type=jnp.float32)
        return (jnp.dot(tri_incl, x, preferred_element_type=jnp.float32)
                + jnp.dot(tot, su, preferred_element_type=jnp.float32))

    eqrank = icumsum(eq.astype(jnp.float32))
    keep = gt | (eq & (eqrank <= m_ties))
    kf = keep.astype(jnp.float32)
    pose = icumsum(kf) - kf

    prow = lax.broadcasted_iota(jnp.int32, (_C, _K), 1).astype(jnp.float32)
    pose2 = pose + (1.0 - kf) * jnp.float32(_N)
    oh_blocks = []
    for r in range(_R):
        oh_blocks.append(
            (pose2[:, r:r + 1] == prow).astype(jnp.bfloat16))
    oh_full = jnp.concatenate(oh_blocks, axis=0)

    iv = lax.broadcasted_iota(jnp.int32, (1, _N), 1)
    civ_hi = (iv >> 8).astype(jnp.float32)
    civ_lo = (iv & 255).astype(jnp.float32)
    srow = (srow0 + b) + jnp.log(mrow_ref[...])
    srow = jnp.maximum(srow, jnp.float32(-3.0e38))

    def split3(x):
        x1 = x.astype(jnp.bfloat16).astype(jnp.float32)
        r1 = x - x1
        x2 = r1.astype(jnp.bfloat16).astype(jnp.float32)
        return x1, x2, r1 - x2

    s1, s2, s3 = split3(srow)
    m1 = mrow_ref[...]
    vals = jnp.concatenate([civ_hi, civ_lo, s1, s2, s3, m1],
                           axis=0).astype(jnp.bfloat16)
    out6 = jnp.dot(vals, oh_full, preferred_element_type=jnp.float32)
    idx_ref[...] = (out6[0:1] * 256.0 + out6[1:2]).astype(jnp.int32)
    s_ref[...] = (out6[2:3] + out6[3:4]) + out6[4:5]
    mo_ref[...] = out6[5:6]


_BLK = 2048

_fused = pl.pallas_call(
    _fused_body,
    grid=(_N // _BLK,),
    in_specs=[pl.BlockSpec((_BLK, _D), lambda i: (i, 0)),
              pl.BlockSpec((_D, 1), lambda i: (0, 0)),
              pl.BlockSpec((1, _N), lambda i: (0, 0)),
              pl.BlockSpec((1, 1), lambda i: (0, 0))],
    out_specs=[pl.BlockSpec((1, _K), lambda i: (0, 0)),
               pl.BlockSpec((1, _K), lambda i: (0, 0)),
               pl.BlockSpec((1, _K), lambda i: (0, 0))],
    out_shape=[jax.ShapeDtypeStruct((1, _K), jnp.int32),
               jax.ShapeDtypeStruct((1, _K), jnp.float32),
               jax.ShapeDtypeStruct((1, _K), jnp.float32)],
    scratch_shapes=[pltpu.VMEM((_N, 1), jnp.float32)],
)


def _gather_body(e_hbm, idx_hbm, emb_out, idxv, rows, sem):
    wid = lax.axis_index("s") * 2 + lax.axis_index("c")
    base = wid * 16
    pltpu.sync_copy(idx_hbm.at[pl.ds(base, 16)], idxv)
    pltpu.async_copy(e_hbm.at[idxv], rows, sem).wait()
    pltpu.sync_copy(rows, emb_out.at[pl.ds(base, 16)])


@functools.cache
def _make_gather():
    return pl.kernel(
        _gather_body,
        mesh=plsc.VectorSubcoreMesh(core_axis_name="c", subcore_axis_name="s"),
        out_type=jax.ShapeDtypeStruct((_K, _D), jnp.float32),
        scratch_types=[pltpu.VMEM((16,), jnp.int32),
                       pltpu.VMEM((16, _D), jnp.float32),
                       pltpu.SemaphoreType.DMA],
    )


def kernel(span_embeddings, spans, span_mask, num_spans_to_keep, W, b):
    del spans, num_spans_to_keep
    e2 = span_embeddings.reshape(_N, _D)
    mrow = span_mask.reshape(1, _N)
    idx2, tops, topm = _fused(e2, W, mrow, b.reshape(1, 1))
    emb = _make_gather()(e2, idx2.reshape(_K))
    return (emb[None], topm, idx2, tops[0][None, :, None])
